# Initial kernel scaffold; baseline (speedup 1.0000x reference)
import functools

import jax
import jax.numpy as jnp
from jax import lax
from jax.experimental import pallas as pl
from jax.experimental.pallas import tpu as pltpu
from jax.experimental.pallas import tpu_sc as plsc

NUM_TOKENS = 16384 * 50
D = 32
NC, NS = 2, 16
NW = NC * NS
PER_W = NUM_TOKENS // NW
G = 128
ROWS_PER_W = PER_W // G
CHUNK_ROWS = 10
NCHUNK = ROWS_PER_W // CHUNK_ROWS
CHUNK = CHUNK_ROWS * G

_mesh = plsc.VectorSubcoreMesh(core_axis_name="c", subcore_axis_name="s")


@functools.partial(
    pl.kernel,
    out_type=jax.ShapeDtypeStruct((NUM_TOKENS, D), jnp.float32),
    mesh=_mesh,
    scratch_types=[
        pltpu.VMEM((ROWS_PER_W, G), jnp.int32),
        pltpu.VMEM((CHUNK, D), jnp.float32),
        pltpu.SemaphoreType.DMA,
    ],
)
def _emb_lookup(idx_hbm, table_hbm, out_hbm, idx_v, buf, gsem):
    wid = lax.axis_index("s") * NC + lax.axis_index("c")
    row0 = wid * ROWS_PER_W
    out0 = wid * PER_W

    pltpu.sync_copy(idx_hbm.at[pl.ds(row0, ROWS_PER_W)], idx_v)

    def chunk_body(c, _):
        handles = []
        for j in range(CHUNK_ROWS):
            r = c * CHUNK_ROWS + j
            handles.append(
                pltpu.async_copy(
                    table_hbm.at[idx_v.at[r]],
                    buf.at[pl.ds(j * G, G)],
                    gsem,
                )
            )
        for h in handles:
            h.wait()
        pltpu.sync_copy(buf, out_hbm.at[pl.ds(out0 + c * CHUNK, CHUNK)])
        return ()

    lax.fori_loop(0, NCHUNK, chunk_body, (), unroll=False)


def kernel(token_ids, embedding):
    flat_idx = token_ids.reshape(NUM_TOKENS // G, G).astype(jnp.int32)
    out = _emb_lookup(flat_idx, embedding)
    return out.reshape(token_ids.shape + (D,))


# speedup vs baseline: 1.1054x; 1.1054x over previous
import functools

import jax
import jax.numpy as jnp
from jax import lax
from jax.experimental import pallas as pl
from jax.experimental.pallas import tpu as pltpu
from jax.experimental.pallas import tpu_sc as plsc

NUM_TOKENS = 16384 * 50
D = 32
NC, NS = 2, 16
NW = NC * NS
PER_W = NUM_TOKENS // NW
G = 128
ROWS_PER_W = PER_W // G
CHUNK_ROWS = 10
NCHUNK = ROWS_PER_W // CHUNK_ROWS
CHUNK = CHUNK_ROWS * G

_mesh = plsc.VectorSubcoreMesh(core_axis_name="c", subcore_axis_name="s")


@functools.partial(
    pl.kernel,
    out_type=jax.ShapeDtypeStruct((NUM_TOKENS, D), jnp.float32),
    mesh=_mesh,
    scratch_types=[
        pltpu.VMEM((ROWS_PER_W, G), jnp.int32),
        pltpu.VMEM((CHUNK, D), jnp.float32),
        pltpu.SemaphoreType.DMA,
    ],
    compiler_params=pltpu.CompilerParams(use_tc_tiling_on_sc=False),
)
def _emb_lookup(idx_hbm, table_hbm, out_hbm, idx_v, buf, gsem):
    wid = lax.axis_index("s") * NC + lax.axis_index("c")
    row0 = wid * ROWS_PER_W
    out0 = wid * PER_W

    pltpu.sync_copy(idx_hbm.at[pl.ds(row0, ROWS_PER_W)], idx_v)

    def chunk_body(c, _):
        handles = []
        for j in range(CHUNK_ROWS):
            r = c * CHUNK_ROWS + j
            handles.append(
                pltpu.async_copy(
                    table_hbm.at[idx_v.at[r]],
                    buf.at[pl.ds(j * G, G)],
                    gsem,
                )
            )
        for h in handles:
            h.wait()
        pltpu.sync_copy(buf, out_hbm.at[pl.ds(out0 + c * CHUNK, CHUNK)])
        return ()

    lax.fori_loop(0, NCHUNK, chunk_body, (), unroll=False)


def kernel(token_ids, embedding):
    flat_idx = token_ids.reshape(NUM_TOKENS // G, G).astype(jnp.int32)
    out = _emb_lookup(flat_idx, embedding)
    return out.reshape(token_ids.shape + (D,))


# speedup vs baseline: 1.1055x; 1.0001x over previous
import functools

import jax
import jax.numpy as jnp
from jax import lax
from jax.experimental import pallas as pl
from jax.experimental.pallas import tpu as pltpu
from jax.experimental.pallas import tpu_sc as plsc

NUM_TOKENS = 16384 * 50
D = 32
NC, NS = 2, 16
NW = NC * NS
PER_W = NUM_TOKENS // NW
G = 128
ROWS_PER_W = PER_W // G
CHUNK_ROWS = 10
NCHUNK = ROWS_PER_W // CHUNK_ROWS
CHUNK = CHUNK_ROWS * G

_mesh = plsc.VectorSubcoreMesh(core_axis_name="c", subcore_axis_name="s")


@functools.partial(
    pl.kernel,
    out_type=jax.ShapeDtypeStruct((NUM_TOKENS, D), jnp.float32),
    mesh=_mesh,
    scratch_types=[
        pltpu.VMEM((PER_W,), jnp.int32),
        pltpu.VMEM((CHUNK, D), jnp.float32),
        pltpu.SemaphoreType.DMA,
    ],
    compiler_params=pltpu.CompilerParams(use_tc_tiling_on_sc=False),
)
def _emb_lookup(idx_hbm, table_hbm, out_hbm, idx_v, buf, gsem):
    wid = lax.axis_index("s") * NC + lax.axis_index("c")
    out0 = wid * PER_W

    pltpu.sync_copy(idx_hbm.at[pl.ds(out0, PER_W)], idx_v)

    def chunk_body(c, _):
        pltpu.async_copy(
            table_hbm.at[idx_v.at[pl.ds(c * CHUNK, CHUNK)]],
            buf,
            gsem,
        ).wait()
        pltpu.sync_copy(buf, out_hbm.at[pl.ds(out0 + c * CHUNK, CHUNK)])
        return ()

    lax.fori_loop(0, NCHUNK, chunk_body, (), unroll=False)


def kernel(token_ids, embedding):
    flat_idx = token_ids.reshape(NUM_TOKENS).astype(jnp.int32)
    out = _emb_lookup(flat_idx, embedding)
    return out.reshape(token_ids.shape + (D,))


# speedup vs baseline: 1.5604x; 1.4114x over previous
import functools

import jax
import jax.numpy as jnp
from jax import lax
from jax.experimental import pallas as pl
from jax.experimental.pallas import tpu as pltpu
from jax.experimental.pallas import tpu_sc as plsc

B, S = 16384, 50
D = 32
V = 1000000
NC, NS = 2, 16
NW = NC * NS
BW = B // NW
G = 128
NBB = BW // G
NGRP = S * NBB
NPAIR = NGRP // 2

_mesh = plsc.VectorSubcoreMesh(core_axis_name="c", subcore_axis_name="s")


def _extract(ids_v, gbuf, ostage, s, bb):
    lane = lax.iota(jnp.int32, 16)
    for i0 in range(0, G, 16):
        t_vec = ids_v[s, pl.ds(bb * G + i0, 16)]
        r32 = (t_vec & 3) << 5
        row = i0 + lane
        for j in range(D):
            col = r32 + j
            vals = plsc.load_gather(gbuf, [row, col])
            ostage[j, pl.ds(i0, 16)] = vals


def _gidx(ids_v, gidx, s, bb):
    for k in range(0, G, 16):
        gidx[pl.ds(k, 16)] = ids_v[s, pl.ds(bb * G + k, 16)] >> 2


@functools.partial(
    pl.kernel,
    out_type=jax.ShapeDtypeStruct((S, D, B), jnp.float32),
    mesh=_mesh,
    scratch_types=[
        pltpu.VMEM((S, BW), jnp.int32),
        pltpu.VMEM((G, 128), jnp.float32),
        pltpu.VMEM((G, 128), jnp.float32),
        pltpu.VMEM((D, G), jnp.float32),
        pltpu.VMEM((D, G), jnp.float32),
        pltpu.VMEM((G,), jnp.int32),
        pltpu.VMEM((G,), jnp.int32),
        pltpu.SemaphoreType.DMA,
        pltpu.SemaphoreType.DMA,
        pltpu.SemaphoreType.DMA,
        pltpu.SemaphoreType.DMA,
    ],
    compiler_params=pltpu.CompilerParams(
        use_tc_tiling_on_sc=True, needs_layout_passes=False),
)
def _emb_lookup(ids_hbm, tab_hbm, out_hbm, ids_v, gbuf0, gbuf1,
                ost0, ost1, gix0, gix1, gsem0, gsem1, ssem0, ssem1):
    wid = lax.axis_index("s") * NC + lax.axis_index("c")
    b0 = wid * BW

    pltpu.sync_copy(ids_hbm.at[:, pl.ds(b0, BW)], ids_v)

    def fire(gidx, gbuf, gsem, s, bb):
        _gidx(ids_v, gidx, s, bb)
        return pltpu.async_copy(tab_hbm.at[gidx], gbuf, gsem)

    fire(gix0, gbuf0, gsem0, 0, 0)

    def pair_body(i, _):
        p = 2 * i
        s_a, bb_a = p // NBB, p % NBB
        s_b, bb_b = (p + 1) // NBB, (p + 1) % NBB
        s_c, bb_c = (p + 2) // NBB, (p + 2) % NBB

        fire(gix1, gbuf1, gsem1, s_b, bb_b)
        pltpu.make_async_copy(tab_hbm.at[gix0], gbuf0, gsem0).wait()

        @pl.when(i > 0)
        def _():
            pltpu.make_async_copy(ost0, out_hbm.at[0, :, pl.ds(0, G)],
                                  ssem0).wait()

        _extract(ids_v, gbuf0, ost0, s_a, bb_a)
        pltpu.async_copy(ost0, out_hbm.at[s_a, :, pl.ds(b0 + bb_a * G, G)],
                         ssem0)

        @pl.when(i < NPAIR - 1)
        def _():
            fire(gix0, gbuf0, gsem0, s_c, bb_c)

        pltpu.make_async_copy(tab_hbm.at[gix1], gbuf1, gsem1).wait()

        @pl.when(i > 0)
        def _():
            pltpu.make_async_copy(ost1, out_hbm.at[0, :, pl.ds(0, G)],
                                  ssem1).wait()

        _extract(ids_v, gbuf1, ost1, s_b, bb_b)
        pltpu.async_copy(ost1, out_hbm.at[s_b, :, pl.ds(b0 + bb_b * G, G)],
                         ssem1)
        return ()

    lax.fori_loop(0, NPAIR, pair_body, (), unroll=False)

    pltpu.make_async_copy(ost0, out_hbm.at[0, :, pl.ds(0, G)], ssem0).wait()
    pltpu.make_async_copy(ost1, out_hbm.at[0, :, pl.ds(0, G)], ssem1).wait()


def kernel(token_ids, embedding):
    ids_t = token_ids.astype(jnp.int32).T
    tab_g = embedding.reshape(V // 4, 128)
    out_p = _emb_lookup(ids_t, tab_g)
    return jnp.transpose(out_p, (2, 0, 1))


# speedup vs baseline: 2.3945x; 1.5345x over previous
import functools

import jax
import jax.numpy as jnp
from jax import lax
from jax.experimental import pallas as pl
from jax.experimental.pallas import tpu as pltpu
from jax.experimental.pallas import tpu_sc as plsc

B, S = 16384, 50
D = 32
V = 1000000
NC, NS = 2, 16
NW = NC * NS
BW = B // NW
G = 128
NBB = BW // G
NGRP = S * NBB
NPAIR = NGRP // 2

_mesh = plsc.VectorSubcoreMesh(core_axis_name="c", subcore_axis_name="s")


def _extract(ids_v, gbuf, ostage, s, bb):
    lane = lax.iota(jnp.int32, 16)

    def chunk(c, _):
        i0 = c * 16
        t_vec = ids_v[s, pl.ds(bb * G + i0, 16)]
        r32 = (t_vec & 3) << 5
        row = i0 + lane
        for half in (0, 16):
            for j0 in range(16):
                jrot = ((j0 + lane) & 15) + half
                vals = plsc.load_gather(gbuf, [row, r32 + jrot])
                plsc.store_scatter(ostage, [jrot, row], vals)
        return ()

    lax.fori_loop(0, G // 16, chunk, (), unroll=False)


def _gidx(ids_v, gidx, s, bb):
    for k in range(0, G, 16):
        gidx[pl.ds(k, 16)] = ids_v[s, pl.ds(bb * G + k, 16)] >> 2


@functools.partial(
    pl.kernel,
    out_type=jax.ShapeDtypeStruct((S, D, B), jnp.float32),
    mesh=_mesh,
    scratch_types=[
        pltpu.VMEM((S, BW), jnp.int32),
        pltpu.VMEM((G, 128), jnp.float32),
        pltpu.VMEM((G, 128), jnp.float32),
        pltpu.VMEM((D, G), jnp.float32),
        pltpu.VMEM((D, G), jnp.float32),
        pltpu.VMEM((G,), jnp.int32),
        pltpu.VMEM((G,), jnp.int32),
        pltpu.SemaphoreType.DMA,
        pltpu.SemaphoreType.DMA,
        pltpu.SemaphoreType.DMA,
        pltpu.SemaphoreType.DMA,
    ],
    compiler_params=pltpu.CompilerParams(
        use_tc_tiling_on_sc=True, needs_layout_passes=False),
)
def _emb_lookup(ids_hbm, tab_hbm, out_hbm, ids_v, gbuf0, gbuf1,
                ost0, ost1, gix0, gix1, gsem0, gsem1, ssem0, ssem1):
    wid = lax.axis_index("s") * NC + lax.axis_index("c")
    b0 = wid * BW

    pltpu.sync_copy(ids_hbm.at[:, pl.ds(b0, BW)], ids_v)

    def fire(gidx, gbuf, gsem, s, bb):
        _gidx(ids_v, gidx, s, bb)
        return pltpu.async_copy(tab_hbm.at[gidx], gbuf, gsem)

    fire(gix0, gbuf0, gsem0, 0, 0)

    def pair_body(i, _):
        p = 2 * i
        s_a, bb_a = p // NBB, p % NBB
        s_b, bb_b = (p + 1) // NBB, (p + 1) % NBB
        s_c, bb_c = (p + 2) // NBB, (p + 2) % NBB

        fire(gix1, gbuf1, gsem1, s_b, bb_b)
        pltpu.make_async_copy(tab_hbm.at[gix0], gbuf0, gsem0).wait()

        @pl.when(i > 0)
        def _():
            pltpu.make_async_copy(ost0, out_hbm.at[0, :, pl.ds(0, G)],
                                  ssem0).wait()

        _extract(ids_v, gbuf0, ost0, s_a, bb_a)
        pltpu.async_copy(ost0, out_hbm.at[s_a, :, pl.ds(b0 + bb_a * G, G)],
                         ssem0)

        @pl.when(i < NPAIR - 1)
        def _():
            fire(gix0, gbuf0, gsem0, s_c, bb_c)

        pltpu.make_async_copy(tab_hbm.at[gix1], gbuf1, gsem1).wait()

        @pl.when(i > 0)
        def _():
            pltpu.make_async_copy(ost1, out_hbm.at[0, :, pl.ds(0, G)],
                                  ssem1).wait()

        _extract(ids_v, gbuf1, ost1, s_b, bb_b)
        pltpu.async_copy(ost1, out_hbm.at[s_b, :, pl.ds(b0 + bb_b * G, G)],
                         ssem1)
        return ()

    lax.fori_loop(0, NPAIR, pair_body, (), unroll=False)

    pltpu.make_async_copy(ost0, out_hbm.at[0, :, pl.ds(0, G)], ssem0).wait()
    pltpu.make_async_copy(ost1, out_hbm.at[0, :, pl.ds(0, G)], ssem1).wait()


def kernel(token_ids, embedding):
    ids_t = token_ids.astype(jnp.int32).T
    tab_g = embedding.reshape(V // 4, 128)
    out_p = _emb_lookup(ids_t, tab_g)
    return jnp.transpose(out_p, (2, 0, 1))


# speedup vs baseline: 2.5388x; 1.0603x over previous
import functools

import jax
import jax.numpy as jnp
from jax import lax
from jax.experimental import pallas as pl
from jax.experimental.pallas import tpu as pltpu
from jax.experimental.pallas import tpu_sc as plsc

B, S = 16384, 50
D = 32
V = 1000000
NC, NS = 2, 16
NW = NC * NS
BW = B // NW
G = 128
NBB = BW // G
NGRP = S * NBB
NPAIR = NGRP // 2

_mesh = plsc.VectorSubcoreMesh(core_axis_name="c", subcore_axis_name="s")


def _extract(ids_v, gbuf, ostage, s, bb):
    lane = lax.iota(jnp.int32, 16)

    def chunk(c, _):
        i0 = c * 16
        t_vec = ids_v[s, pl.ds(bb * G + i0, 16)]
        r32 = (t_vec & 3) << 5
        row = i0 + lane
        for half in (0, 16):
            for j0 in range(16):
                jrot = ((j0 + lane) & 15) + half
                vals = plsc.load_gather(gbuf, [row, r32 + jrot])
                plsc.store_scatter(ostage, [jrot, row], vals)
        return ()

    lax.fori_loop(0, G // 16, chunk, (), unroll=False)


def _gidx(ids_v, gidx, s, bb):
    for k in range(0, G, 16):
        gidx[pl.ds(k, 16)] = ids_v[s, pl.ds(bb * G + k, 16)] >> 2


@functools.partial(
    pl.kernel,
    out_type=jax.ShapeDtypeStruct((S, D, B), jnp.float32),
    mesh=_mesh,
    scratch_types=[
        pltpu.VMEM((S, BW), jnp.int32),
        [pltpu.VMEM((G, 128), jnp.float32) for _ in range(4)],
        [pltpu.VMEM((D, G), jnp.float32) for _ in range(2)],
        [pltpu.VMEM((G,), jnp.int32) for _ in range(4)],
        [pltpu.SemaphoreType.DMA for _ in range(4)],
        [pltpu.SemaphoreType.DMA for _ in range(2)],
    ],
    compiler_params=pltpu.CompilerParams(
        use_tc_tiling_on_sc=True, needs_layout_passes=False),
)
def _emb_lookup(ids_hbm, tab_hbm, out_hbm, ids_v, gbuf, ost, gix, gsem, ssem):
    wid = lax.axis_index("s") * NC + lax.axis_index("c")
    b0 = wid * BW

    pltpu.sync_copy(ids_hbm.at[:, pl.ds(b0, BW)], ids_v)

    def fire(k, s):
        _gidx(ids_v, gix[k], s, k)
        pltpu.async_copy(tab_hbm.at[gix[k]], gbuf[k], gsem[k])

    for k in range(NBB):
        fire(k, 0)

    def row_body(s, _):
        for k in range(NBB):
            pltpu.make_async_copy(tab_hbm.at[gix[k]], gbuf[k],
                                  gsem[k]).wait()
            if k >= 2:
                pltpu.make_async_copy(ost[k % 2],
                                      out_hbm.at[0, :, pl.ds(0, G)],
                                      ssem[k % 2]).wait()
            else:
                @pl.when(s > 0)
                def _():
                    pltpu.make_async_copy(ost[k % 2],
                                          out_hbm.at[0, :, pl.ds(0, G)],
                                          ssem[k % 2]).wait()

            _extract(ids_v, gbuf[k], ost[k % 2], s, k)
            pltpu.async_copy(ost[k % 2],
                             out_hbm.at[s, :, pl.ds(b0 + k * G, G)],
                             ssem[k % 2])

            @pl.when(s < S - 1)
            def _():
                fire(k, s + 1)
        return ()

    lax.fori_loop(0, S, row_body, (), unroll=False)

    pltpu.make_async_copy(ost[0], out_hbm.at[0, :, pl.ds(0, G)], ssem[0]).wait()
    pltpu.make_async_copy(ost[1], out_hbm.at[0, :, pl.ds(0, G)], ssem[1]).wait()


def kernel(token_ids, embedding):
    ids_t = token_ids.astype(jnp.int32).T
    tab_g = embedding.reshape(V // 4, 128)
    out_p = _emb_lookup(ids_t, tab_g)
    return jnp.transpose(out_p, (2, 0, 1))


# speedup vs baseline: 2.5415x; 1.0011x over previous
import functools

import jax
import jax.numpy as jnp
from jax import lax
from jax.experimental import pallas as pl
from jax.experimental.pallas import tpu as pltpu
from jax.experimental.pallas import tpu_sc as plsc

B, S = 16384, 50
D = 32
V = 1000000
NC, NS = 2, 16
NW = NC * NS
BW = B // NW
G = 128
NBB = BW // G
NGRP = S * NBB
NPAIR = NGRP // 2

_mesh = plsc.VectorSubcoreMesh(core_axis_name="c", subcore_axis_name="s")


def _extract(ids_v, gbuf, ostage, s, bb):
    lane = lax.iota(jnp.int32, 16)

    def chunk(c, _):
        i0 = c * 16
        t_vec = ids_v[s, pl.ds(bb * G + i0, 16)]
        r32 = (t_vec & 3) << 5
        row = i0 + lane
        for half in (0, 16):
            for j0 in range(16):
                jrot = ((j0 + lane) & 15) + half
                vals = plsc.load_gather(gbuf, [row, r32 + jrot])
                plsc.store_scatter(ostage, [jrot, row], vals)
        return ()

    lax.fori_loop(0, G // 16, chunk, (), unroll=False)


def _gidx(ids_v, gidx, s, bb):
    for k in range(0, G, 16):
        gidx[pl.ds(k, 16)] = ids_v[s, pl.ds(bb * G + k, 16)] >> 2


@functools.partial(
    pl.kernel,
    out_type=jax.ShapeDtypeStruct((S, D, B), jnp.float32),
    mesh=_mesh,
    scratch_types=[
        pltpu.VMEM((S, BW), jnp.int32),
        [pltpu.VMEM((G, 128), jnp.float32) for _ in range(4)],
        [pltpu.VMEM((D, G), jnp.float32) for _ in range(2)],
        [pltpu.VMEM((G,), jnp.int32) for _ in range(4)],
        [pltpu.SemaphoreType.DMA for _ in range(4)],
        [pltpu.SemaphoreType.DMA for _ in range(2)],
    ],
    compiler_params=pltpu.CompilerParams(
        use_tc_tiling_on_sc=True, needs_layout_passes=False,
        disable_bounds_checks=True, skip_device_barrier=True),
)
def _emb_lookup(ids_hbm, tab_hbm, out_hbm, ids_v, gbuf, ost, gix, gsem, ssem):
    wid = lax.axis_index("s") * NC + lax.axis_index("c")
    b0 = wid * BW

    pltpu.sync_copy(ids_hbm.at[:, pl.ds(b0, BW)], ids_v)

    def fire(k, s):
        _gidx(ids_v, gix[k], s, k)
        pltpu.async_copy(tab_hbm.at[gix[k]], gbuf[k], gsem[k])

    for k in range(NBB):
        fire(k, 0)

    def row_body(s, _):
        for k in range(NBB):
            pltpu.make_async_copy(tab_hbm.at[gix[k]], gbuf[k],
                                  gsem[k]).wait()
            if k >= 2:
                pltpu.make_async_copy(ost[k % 2],
                                      out_hbm.at[0, :, pl.ds(0, G)],
                                      ssem[k % 2]).wait()
            else:
                @pl.when(s > 0)
                def _():
                    pltpu.make_async_copy(ost[k % 2],
                                          out_hbm.at[0, :, pl.ds(0, G)],
                                          ssem[k % 2]).wait()

            _extract(ids_v, gbuf[k], ost[k % 2], s, k)
            pltpu.async_copy(ost[k % 2],
                             out_hbm.at[s, :, pl.ds(b0 + k * G, G)],
                             ssem[k % 2])

            @pl.when(s < S - 1)
            def _():
                fire(k, s + 1)
        return ()

    lax.fori_loop(0, S, row_body, (), unroll=False)

    pltpu.make_async_copy(ost[0], out_hbm.at[0, :, pl.ds(0, G)], ssem[0]).wait()
    pltpu.make_async_copy(ost[1], out_hbm.at[0, :, pl.ds(0, G)], ssem[1]).wait()


def kernel(token_ids, embedding):
    ids_t = token_ids.astype(jnp.int32).T
    tab_g = embedding.reshape(V // 4, 128)
    out_p = _emb_lookup(ids_t, tab_g)
    return jnp.transpose(out_p, (2, 0, 1))
